# baseline (speedup 1.0000x reference)
import functools

import jax
import jax.numpy as jnp
from jax import lax
from jax.experimental import pallas as pl
from jax.experimental.pallas import tpu as pltpu
from jax.experimental.pallas import tpu_sc as plsc

_NUM_FIELDS = 26
_FIELD_SIZE = 100000
_BATCH = 16384
_NC = 2
_NS = 16
_NW = _NC * _NS
_BPW = _BATCH // _NW
_CHUNK = 128
_NBLK = _BPW // _CHUNK
_NCHUNK = _NBLK * _NUM_FIELDS
_NSL = _CHUNK // 16
_XPW = _BPW * _NUM_FIELDS

_MESH = plsc.VectorSubcoreMesh(
    core_axis_name="c", subcore_axis_name="s",
    num_cores=_NC, num_subcores=_NS,
)
_PARAMS = pltpu.CompilerParams(needs_layout_passes=False)


@functools.partial(
    pl.kernel,
    mesh=_MESH,
    out_type=jax.ShapeDtypeStruct((_NW, _NCHUNK, _CHUNK), jnp.int32),
    scratch_types=[
        pltpu.VMEM((_BPW, _NUM_FIELDS), jnp.int32),
        pltpu.VMEM((_NCHUNK, _CHUNK), jnp.int32),
    ],
    compiler_params=_PARAMS,
)
def _build_indices(x_hbm, idx_hbm, x_v, idx_v):
    wid = lax.axis_index("s") * _NC + lax.axis_index("c")
    pltpu.sync_copy(x_hbm.at[pl.ds(wid * _BPW, _BPW), :], x_v)

    lanes = jnp.arange(16, dtype=jnp.int32)
    zeros16 = jnp.zeros((16,), jnp.int32)

    for c in range(_NBLK):
        def build(f, _, c=c):
            j = c * _NUM_FIELDS + f
            off = f * _FIELD_SIZE
            cols = zeros16 + f
            for s in range(_NSL):
                rows = (c * _CHUNK + s * 16) + lanes
                vals = plsc.load_gather(x_v, [rows, cols])
                idx_v[j, pl.ds(s * 16, 16)] = vals + off
            return 0

        lax.fori_loop(0, _NUM_FIELDS, build, 0)

    pltpu.sync_copy(idx_v, idx_hbm.at[wid])


@functools.partial(
    pl.kernel,
    mesh=_MESH,
    out_type=jax.ShapeDtypeStruct((_BATCH,), jnp.float32),
    scratch_types=[
        pltpu.VMEM((_NCHUNK, _CHUNK), jnp.int32),
        pltpu.VMEM((_NCHUNK, _CHUNK), jnp.float32),
        pltpu.VMEM((_BPW,), jnp.float32),
        pltpu.VMEM((16,), jnp.float32),
        pltpu.SemaphoreType.DMA,
    ],
    compiler_params=_PARAMS,
)
def _gather_sum(idx_hbm, table_hbm, bias_hbm, out_hbm, idx_v, val_v, out_v,
                bias_v, sem):
    wid = lax.axis_index("s") * _NC + lax.axis_index("c")
    pltpu.sync_copy(idx_hbm.at[wid], idx_v)
    pltpu.sync_copy(bias_hbm, bias_v)

    def fire(j, _):
        pltpu.async_copy(table_hbm.at[idx_v.at[j]], val_v.at[j], sem)
        return 0

    lax.fori_loop(0, _NCHUNK, fire, 0)

    def drain(j, _):
        pltpu.make_async_copy(
            table_hbm.at[idx_v.at[j]], val_v.at[j], sem
        ).wait()
        return 0

    lax.fori_loop(0, _NCHUNK, drain, 0)

    bias = bias_v[pl.ds(0, 16)]
    for c in range(_NBLK):
        def body(f, acc, c=c):
            j = c * _NUM_FIELDS + f
            row = val_v.at[j]
            return tuple(
                acc[s] + row[pl.ds(s * 16, 16)] for s in range(_NSL)
            )

        zeros = tuple(jnp.zeros((16,), jnp.float32) for _ in range(_NSL))
        acc = lax.fori_loop(0, _NUM_FIELDS, body, zeros)
        for s in range(_NSL):
            out_v[pl.ds(c * _CHUNK + s * 16, 16)] = acc[s] + bias

    pltpu.sync_copy(out_v, out_hbm.at[pl.ds(wid * _BPW, _BPW)])


def kernel(x, fc_weight, bias):
    idx = _build_indices(x.astype(jnp.int32))
    table = fc_weight.reshape(-1)
    bias16 = jnp.broadcast_to(bias, (16,))
    out = _gather_sum(idx, table, bias16)
    return out.reshape(_BATCH, 1)


# speedup vs baseline: 1.0074x; 1.0074x over previous
import functools

import jax
import jax.numpy as jnp
from jax import lax
from jax.experimental import pallas as pl
from jax.experimental.pallas import tpu as pltpu
from jax.experimental.pallas import tpu_sc as plsc

_NUM_FIELDS = 26
_FIELD_SIZE = 100000
_BATCH = 16384
_NC = 2
_NS = 16
_NW = _NC * _NS
_BPW = _BATCH // _NW
_CHUNK = 128
_NBLK = _BPW // _CHUNK
_NCHUNK = _NBLK * _NUM_FIELDS
_NSL = _CHUNK // 16
_XPW = _BPW * _NUM_FIELDS

_MESH = plsc.VectorSubcoreMesh(
    core_axis_name="c", subcore_axis_name="s",
    num_cores=_NC, num_subcores=_NS,
)
_PARAMS = pltpu.CompilerParams(needs_layout_passes=False)


@functools.partial(
    pl.kernel,
    mesh=_MESH,
    out_type=jax.ShapeDtypeStruct((_NW, _NCHUNK, _CHUNK), jnp.int32),
    scratch_types=[
        pltpu.VMEM((_BPW, _NUM_FIELDS), jnp.int32),
        pltpu.VMEM((_NCHUNK, _CHUNK), jnp.int32),
    ],
    compiler_params=_PARAMS,
)
def _build_indices(x_hbm, idx_hbm, x_v, idx_v):
    wid = lax.axis_index("s") * _NC + lax.axis_index("c")
    pltpu.sync_copy(x_hbm.at[pl.ds(wid * _BPW, _BPW), :], x_v)

    lanes = jnp.arange(16, dtype=jnp.int32)
    zeros16 = jnp.zeros((16,), jnp.int32)

    for c in range(_NBLK):
        def build(f, _, c=c):
            j = c * _NUM_FIELDS + f
            off = f * _FIELD_SIZE
            cols = zeros16 + f
            for s in range(_NSL):
                rows = (c * _CHUNK + s * 16) + lanes
                vals = plsc.load_gather(x_v, [rows, cols])
                idx_v[j, pl.ds(s * 16, 16)] = vals + off
            return 0

        lax.fori_loop(0, _NUM_FIELDS, build, 0)

    pltpu.sync_copy(idx_v, idx_hbm.at[wid])


@functools.partial(
    pl.kernel,
    mesh=_MESH,
    out_type=jax.ShapeDtypeStruct((_BATCH,), jnp.float32),
    scratch_types=[
        pltpu.VMEM((_NCHUNK, _CHUNK), jnp.int32),
        pltpu.VMEM((_NCHUNK, _CHUNK), jnp.float32),
        pltpu.VMEM((_BPW,), jnp.float32),
        pltpu.SemaphoreType.DMA,
    ],
    compiler_params=_PARAMS,
)
def _gather_sum(idx_hbm, table_hbm, out_hbm, idx_v, val_v, out_v, sem):
    wid = lax.axis_index("s") * _NC + lax.axis_index("c")
    pltpu.sync_copy(idx_hbm.at[wid], idx_v)

    def fire(j, _):
        pltpu.async_copy(table_hbm.at[idx_v.at[j]], val_v.at[j], sem)
        return 0

    lax.fori_loop(0, _NCHUNK, fire, 0)

    def drain(j, _):
        pltpu.make_async_copy(
            table_hbm.at[idx_v.at[j]], val_v.at[j], sem
        ).wait()
        return 0

    lax.fori_loop(0, _NCHUNK, drain, 0)

    for c in range(_NBLK):
        def body(f, acc, c=c):
            j = c * _NUM_FIELDS + f
            row = val_v.at[j]
            return tuple(
                acc[s] + row[pl.ds(s * 16, 16)] for s in range(_NSL)
            )

        zeros = tuple(jnp.zeros((16,), jnp.float32) for _ in range(_NSL))
        acc = lax.fori_loop(0, _NUM_FIELDS, body, zeros)
        for s in range(_NSL):
            out_v[pl.ds(c * _CHUNK + s * 16, 16)] = acc[s]

    pltpu.sync_copy(out_v, out_hbm.at[pl.ds(wid * _BPW, _BPW)])


def kernel(x, fc_weight, bias):
    idx = _build_indices(x.astype(jnp.int32))
    table = fc_weight.reshape(-1)
    out = _gather_sum(idx, table)
    return out.reshape(_BATCH, 1) + bias[None, :]
